# baseline (speedup 1.0000x reference)
import dataclasses
import functools

import jax
import jax.numpy as jnp
from jax import lax
from jax.experimental import pallas as pl
from jax.experimental.pallas import tpu as pltpu
from jax.experimental.pallas import tpu_sc as plsc

_VOCAB = 1000000
_EMB = 32
_SEQ = 200
_BATCH = 4096
_NW = 32
_BPW = _BATCH // _NW
_CH = 10
_RING = 2 * _CH
_NCH = _SEQ // _CH

_mesh = plsc.VectorSubcoreMesh(core_axis_name="c", subcore_axis_name="s")

_cp = pltpu.CompilerParams()
for _f, _v in (("needs_layout_passes", False), ("use_tc_tiling_on_sc", False)):
    if _f in pltpu.CompilerParams.__dataclass_fields__:
        _cp = dataclasses.replace(_cp, **{_f: _v})


@functools.partial(
    pl.kernel,
    out_type=jax.ShapeDtypeStruct((_BATCH,), jnp.float32),
    mesh=_mesh,
    compiler_params=_cp,
    scratch_types=[
        pltpu.VMEM((_SEQ, _BPW), jnp.int32),
        pltpu.VMEM((_RING, _BPW, _EMB), jnp.float32),
        pltpu.VMEM((_BPW,), jnp.float32),
        pltpu.VMEM((40,), jnp.float32),
        pltpu.SemaphoreType.DMA,
        pltpu.SemaphoreType.DMA,
    ],
)
def _sc_fused(text_hbm, table_hbm, fcp_hbm, out_hbm,
              idx_v, ring_v, res_v, fc_s, sem_a, sem_b):
    wid = lax.axis_index("s") * 2 + lax.axis_index("c")
    base = wid * _BPW
    pltpu.sync_copy(fcp_hbm, fc_s)
    pltpu.sync_copy(text_hbm.at[:, pl.ds(base, _BPW)], idx_v)

    iota = lax.iota(jnp.int32, 16)
    fcvecs = [plsc.load_gather(fc_s, [jnp.full((16,), e, jnp.int32)])
              for e in range(_EMB)]
    bias = plsc.load_gather(fc_s, [jnp.full((16,), _EMB, jnp.int32)])

    def fire(s, slot, sem):
        pltpu.async_copy(table_hbm.at[idx_v.at[s]], ring_v.at[slot], sem)

    def fire_chunk(c, half, sem):
        for j in range(_CH):
            fire(c * _CH + j, half + j, sem)

    def drain_chunk(half, sem):
        for j in range(_CH):
            pltpu.make_async_copy(
                table_hbm.at[idx_v.at[j]], ring_v.at[half + j], sem,
            ).wait()

    def acc_chunk(half, accs):
        def row_body(j, accs):
            new = list(accs)
            slot = jnp.full((16,), half + j, jnp.int32)
            for g in range(8):
                a = new[g]
                rows = iota + g * 16
                for e in range(_EMB):
                    col = jnp.full((16,), e, jnp.int32)
                    vals = plsc.load_gather(ring_v, [slot, rows, col])
                    a = a + vals * fcvecs[e]
                new[g] = a
            return tuple(new)

        return lax.fori_loop(0, _CH, row_body, accs)

    def chunk_body(c, accs):
        for j in range(_CH):
            pltpu.sync_copy(table_hbm.at[idx_v.at[c * _CH + j]], ring_v.at[j])
        new = list(accs)
        for j in range(_CH):
            row_ref = ring_v.at[j]
            for g in range(8):
                rows = iota + g * 16

                def e_body(e, a, rows=rows, row_ref=row_ref):
                    col = jnp.full((16,), e, jnp.int32)
                    fcv = plsc.load_gather(fc_s, [col])
                    vals = plsc.load_gather(row_ref, [rows, col])
                    return a + vals * fcv

                new[g] = lax.fori_loop(0, _EMB, e_body, new[g])
        return tuple(new)

    accs = lax.fori_loop(
        0, _NCH, chunk_body,
        tuple(jnp.zeros((16,), jnp.float32) for _ in range(8)),
    )

    for g in range(8):
        res_v[pl.ds(g * 16, 16)] = accs[g] * (1.0 / _SEQ) + bias
    pltpu.sync_copy(res_v, out_hbm.at[pl.ds(base, _BPW)])


def kernel(text, table, fc_w, fc_b):
    fcp = jnp.zeros((40,), jnp.float32)
    fcp = fcp.at[:_EMB].set(fc_w.reshape(-1).astype(jnp.float32))
    fcp = fcp.at[_EMB].set(fc_b.reshape(())[()].astype(jnp.float32))
    out = _sc_fused(text, table, fcp)
    return out.reshape(_BATCH, 1)


# speedup vs baseline: 1.2594x; 1.2594x over previous
import dataclasses
import functools

import jax
import jax.numpy as jnp
from jax import lax
from jax.experimental import pallas as pl
from jax.experimental.pallas import tpu as pltpu
from jax.experimental.pallas import tpu_sc as plsc

_VOCAB = 1000000
_EMB = 32
_SEQ = 200
_BATCH = 4096
_NW = 32
_BPW = _BATCH // _NW
_CH = 10
_RING = 2 * _CH
_NCH = _SEQ // _CH

_mesh = plsc.VectorSubcoreMesh(core_axis_name="c", subcore_axis_name="s")

_cp = pltpu.CompilerParams()
for _f, _v in (("needs_layout_passes", False), ("use_tc_tiling_on_sc", False)):
    if _f in pltpu.CompilerParams.__dataclass_fields__:
        _cp = dataclasses.replace(_cp, **{_f: _v})


@functools.partial(
    pl.kernel,
    out_type=jax.ShapeDtypeStruct((_BATCH,), jnp.float32),
    mesh=_mesh,
    compiler_params=_cp,
    scratch_types=[
        pltpu.VMEM((_SEQ, _BPW), jnp.int32),
        pltpu.VMEM((_RING, _BPW, _EMB), jnp.float32),
        pltpu.VMEM((_BPW,), jnp.float32),
        pltpu.VMEM((40,), jnp.float32),
        pltpu.SemaphoreType.DMA,
        pltpu.SemaphoreType.DMA,
    ],
)
def _sc_fused(text_hbm, table_hbm, fcp_hbm, out_hbm,
              idx_v, ring_v, res_v, fc_s, sem_a, sem_b):
    wid = lax.axis_index("s") * 2 + lax.axis_index("c")
    base = wid * _BPW
    pltpu.sync_copy(fcp_hbm, fc_s)
    pltpu.sync_copy(text_hbm.at[:, pl.ds(base, _BPW)], idx_v)

    iota = lax.iota(jnp.int32, 16)

    def fire(s, slot, sem):
        pltpu.async_copy(table_hbm.at[idx_v.at[s]], ring_v.at[slot], sem)

    def fire_chunk(c, half, sem):
        for j in range(_CH):
            fire(c * _CH + j, half + j, sem)

    def drain_chunk(half, sem):
        for j in range(_CH):
            pltpu.make_async_copy(
                table_hbm.at[idx_v.at[j]], ring_v.at[half + j], sem,
            ).wait()

    def acc_chunk(half, accs):
        for j in range(_CH):
            row_ref = ring_v.at[half + j]

            def e_body(e, accs8, row_ref=row_ref):
                col = jnp.full((16,), e, jnp.int32)
                fcv = plsc.load_gather(fc_s, [col])
                return tuple(
                    accs8[g]
                    + plsc.load_gather(row_ref, [iota + g * 16, col]) * fcv
                    for g in range(8)
                )

            accs = lax.fori_loop(0, _EMB, e_body, accs)
        return accs

    fire_chunk(0, 0, sem_a)

    def pair_body(k, accs):
        fire_chunk(2 * k + 1, _CH, sem_b)
        drain_chunk(0, sem_a)
        accs = acc_chunk(0, accs)

        @pl.when(k + 1 < _NCH // 2)
        def _():
            fire_chunk(2 * k + 2, 0, sem_a)

        drain_chunk(_CH, sem_b)
        return acc_chunk(_CH, accs)

    accs = lax.fori_loop(
        0, _NCH // 2, pair_body,
        tuple(jnp.zeros((16,), jnp.float32) for _ in range(8)),
    )

    bias = plsc.load_gather(fc_s, [jnp.full((16,), _EMB, jnp.int32)])
    for g in range(8):
        res_v[pl.ds(g * 16, 16)] = accs[g] * (1.0 / _SEQ) + bias
    pltpu.sync_copy(res_v, out_hbm.at[pl.ds(base, _BPW)])


def kernel(text, table, fc_w, fc_b):
    fcp = jnp.zeros((40,), jnp.float32)
    fcp = fcp.at[:_EMB].set(fc_w.reshape(-1).astype(jnp.float32))
    fcp = fcp.at[_EMB].set(fc_b.reshape(())[()].astype(jnp.float32))
    out = _sc_fused(text, table, fcp)
    return out.reshape(_BATCH, 1)


# speedup vs baseline: 1.7859x; 1.4181x over previous
import dataclasses
import functools

import jax
import jax.numpy as jnp
from jax import lax
from jax.experimental import pallas as pl
from jax.experimental.pallas import tpu as pltpu
from jax.experimental.pallas import tpu_sc as plsc

_VOCAB = 1000000
_EMB = 32
_SEQ = 200
_BATCH = 4096
_NW = 32
_BPW = _BATCH // _NW
_CH = 10
_RING = 2 * _CH
_NCH = _SEQ // _CH

_mesh = plsc.VectorSubcoreMesh(core_axis_name="c", subcore_axis_name="s")

_cp = pltpu.CompilerParams()
for _f, _v in (("needs_layout_passes", False), ("use_tc_tiling_on_sc", False)):
    if _f in pltpu.CompilerParams.__dataclass_fields__:
        _cp = dataclasses.replace(_cp, **{_f: _v})


@functools.partial(
    pl.kernel,
    out_type=jax.ShapeDtypeStruct((_BATCH,), jnp.float32),
    mesh=_mesh,
    compiler_params=_cp,
    scratch_types=[
        pltpu.VMEM((_SEQ, _BPW), jnp.int32),
        pltpu.VMEM((_RING, _BPW, _EMB), jnp.float32),
        pltpu.VMEM((_BPW,), jnp.float32),
        pltpu.VMEM((_BPW * 16,), jnp.float32),
        pltpu.VMEM((40,), jnp.float32),
        pltpu.SemaphoreType.DMA,
        pltpu.SemaphoreType.DMA,
    ],
)
def _sc_fused(text_hbm, table_hbm, fcp_hbm, out_hbm,
              idx_v, ring_v, res_v, acc_f, fc_s, sem_a, sem_b):
    wid = lax.axis_index("s") * 2 + lax.axis_index("c")
    base = wid * _BPW
    pltpu.sync_copy(fcp_hbm, fc_s)
    pltpu.sync_copy(text_hbm.at[:, pl.ds(base, _BPW)], idx_v)

    iota = lax.iota(jnp.int32, 16)

    def fire(s, slot, sem):
        pltpu.async_copy(table_hbm.at[idx_v.at[s]], ring_v.at[slot], sem)

    def fire_chunk(c, half, sem):
        for j in range(_CH):
            fire(c * _CH + j, half + j, sem)

    def drain_chunk(half, sem):
        for j in range(_CH):
            pltpu.make_async_copy(
                table_hbm.at[idx_v.at[j]], ring_v.at[half + j], sem,
            ).wait()

    fcA = fc_s[pl.ds(0, 16)]
    fcB = fc_s[pl.ds(16, 16)]
    zero16 = jnp.zeros((16,), jnp.float32)

    def zero_body(i, _):
        acc_f[pl.ds(i * 16, 16)] = zero16
        return 0

    lax.fori_loop(0, _BPW, zero_body, 0)

    def acc_chunk(half):
        for j in range(_CH):
            def b_body(b4, _, j=j):
                for u in range(4):
                    b = b4 * 4 + u
                    vA = ring_v[half + j, b, pl.ds(0, 16)]
                    vB = ring_v[half + j, b, pl.ds(16, 16)]
                    a = acc_f[pl.ds(b * 16, 16)]
                    acc_f[pl.ds(b * 16, 16)] = a + vA * fcA + vB * fcB
                return 0

            lax.fori_loop(0, _BPW // 4, b_body, 0)

    fire_chunk(0, 0, sem_a)

    def pair_body(k, _):
        fire_chunk(2 * k + 1, _CH, sem_b)
        drain_chunk(0, sem_a)
        acc_chunk(0)

        @pl.when(k + 1 < _NCH // 2)
        def _():
            fire_chunk(2 * k + 2, 0, sem_a)

        drain_chunk(_CH, sem_b)
        acc_chunk(_CH)
        return 0

    lax.fori_loop(0, _NCH // 2, pair_body, 0)

    bias = plsc.load_gather(fc_s, [jnp.full((16,), _EMB, jnp.int32)])
    for g in range(8):
        a = zero16
        for l in range(16):
            a = a + plsc.load_gather(acc_f, [(iota + g * 16) * 16 + l])
        res_v[pl.ds(g * 16, 16)] = a * (1.0 / _SEQ) + bias
    pltpu.sync_copy(res_v, out_hbm.at[pl.ds(base, _BPW)])


def kernel(text, table, fc_w, fc_b):
    fcp = jnp.zeros((40,), jnp.float32)
    fcp = fcp.at[:_EMB].set(fc_w.reshape(-1).astype(jnp.float32))
    fcp = fcp.at[_EMB].set(fc_b.reshape(())[()].astype(jnp.float32))
    out = _sc_fused(text, table, fcp)
    return out.reshape(_BATCH, 1)


# speedup vs baseline: 2.1211x; 1.1877x over previous
import dataclasses
import functools

import jax
import jax.numpy as jnp
from jax import lax
from jax.experimental import pallas as pl
from jax.experimental.pallas import tpu as pltpu
from jax.experimental.pallas import tpu_sc as plsc

_VOCAB = 1000000
_EMB = 32
_SEQ = 200
_BATCH = 4096
_NW = 32
_BPW = _BATCH // _NW
_CH = 10
_RING = 2 * _CH
_NCH = _SEQ // _CH

_mesh = plsc.VectorSubcoreMesh(core_axis_name="c", subcore_axis_name="s")

_cp = pltpu.CompilerParams()
for _f, _v in (("needs_layout_passes", False), ("use_tc_tiling_on_sc", False)):
    if _f in pltpu.CompilerParams.__dataclass_fields__:
        _cp = dataclasses.replace(_cp, **{_f: _v})


@functools.partial(
    pl.kernel,
    out_type=jax.ShapeDtypeStruct((_BATCH,), jnp.float32),
    mesh=_mesh,
    compiler_params=_cp,
    scratch_types=[
        pltpu.VMEM((_SEQ, _BPW), jnp.int32),
        pltpu.VMEM((_RING, _BPW, _EMB), jnp.float32),
        pltpu.VMEM((_BPW,), jnp.float32),
        pltpu.VMEM((_BPW * 16,), jnp.float32),
        pltpu.VMEM((40,), jnp.float32),
        pltpu.SemaphoreType.DMA,
        pltpu.SemaphoreType.DMA,
    ],
)
def _sc_fused(text_hbm, table_hbm, fcp_hbm, out_hbm,
              idx_v, ring_v, res_v, acc_f, fc_s, sem_a, sem_b):
    wid = lax.axis_index("s") * 2 + lax.axis_index("c")
    base = wid * _BPW
    pltpu.sync_copy(fcp_hbm, fc_s)
    pltpu.sync_copy(text_hbm.at[:, pl.ds(base, _BPW)], idx_v)

    iota = lax.iota(jnp.int32, 16)

    def fire(s, slot, sem):
        pltpu.async_copy(table_hbm.at[idx_v.at[s]], ring_v.at[slot], sem)

    def fire_chunk(c, half, sem):
        for j in range(_CH):
            fire(c * _CH + j, half + j, sem)

    def drain_chunk(half, sem):
        for j in range(_CH):
            pltpu.make_async_copy(
                table_hbm.at[idx_v.at[j]], ring_v.at[half + j], sem,
            ).wait()

    fcA = fc_s[pl.ds(0, 16)]
    fcB = fc_s[pl.ds(16, 16)]
    zero16 = jnp.zeros((16,), jnp.float32)

    def zero_body(i, _):
        acc_f[pl.ds(i * 16, 16)] = zero16
        return 0

    lax.fori_loop(0, _BPW, zero_body, 0)

    def acc_chunk(half):
        def b_body(b4, _):
            for u in range(4):
                b = b4 * 4 + u
                a = acc_f[pl.ds(b * 16, 16)]
                for j in range(_CH):
                    vA = ring_v[half + j, b, pl.ds(0, 16)]
                    vB = ring_v[half + j, b, pl.ds(16, 16)]
                    a = a + vA * fcA + vB * fcB
                acc_f[pl.ds(b * 16, 16)] = a
            return 0

        lax.fori_loop(0, _BPW // 4, b_body, 0)

    fire_chunk(0, 0, sem_a)

    def pair_body(k, _):
        fire_chunk(2 * k + 1, _CH, sem_b)
        drain_chunk(0, sem_a)
        acc_chunk(0)

        @pl.when(k + 1 < _NCH // 2)
        def _():
            fire_chunk(2 * k + 2, 0, sem_a)

        drain_chunk(_CH, sem_b)
        acc_chunk(_CH)
        return 0

    lax.fori_loop(0, _NCH // 2, pair_body, 0)

    bias = plsc.load_gather(fc_s, [jnp.full((16,), _EMB, jnp.int32)])
    for g in range(8):
        a = zero16
        for l in range(16):
            a = a + plsc.load_gather(acc_f, [(iota + g * 16) * 16 + l])
        res_v[pl.ds(g * 16, 16)] = a * (1.0 / _SEQ) + bias
    pltpu.sync_copy(res_v, out_hbm.at[pl.ds(base, _BPW)])


def kernel(text, table, fc_w, fc_b):
    fcp = jnp.zeros((40,), jnp.float32)
    fcp = fcp.at[:_EMB].set(fc_w.reshape(-1).astype(jnp.float32))
    fcp = fcp.at[_EMB].set(fc_b.reshape(())[()].astype(jnp.float32))
    out = _sc_fused(text, table, fcp)
    return out.reshape(_BATCH, 1)
